# baseline (speedup 1.0000x reference)
import functools

import jax
import jax.numpy as jnp
from jax import lax
from jax.experimental import pallas as pl
from jax.experimental.pallas import tpu as pltpu
from jax.experimental.pallas import tpu_sc as plsc

N = 10000
E = 320000
D = 128
DH = D // 2
LANES = 128
NC = 2
NS = 16
NW = NC * NS
E_ROWS = -(-E // LANES)
E_ROWS_PAD = -(-E_ROWS // (NW * 8)) * NW * 8
R_PER_W = E_ROWS_PAD // NW
R_PER_T = E_ROWS_PAD // NS
N_PAD = 10112
RPT = N_PAD // NS
IDX_CH = 16
IDX_CH_DEG = 8
BLK = N_PAD // 16
BN_RSQRT = 1.0 / (1.0 + 1e-5) ** 0.5

_MESH = plsc.VectorSubcoreMesh(core_axis_name="c", subcore_axis_name="s")


def _sc_segsum_body(with_deg, IDX_CH, *refs):
    if with_deg:
        (y_hbm, src_hbm, dst_hbm, zd_hbm, z16_hbm, ones_hbm,
         acc_out, deg_out, y_sh, acc_sh, deg_sh, src_v, dst_vA, dst_vB,
         r0, r1, r2, r3, g0, g1, g2, g3, s0, s1, s2, s3, ones_v,
         d0, d1) = refs
    else:
        (y_hbm, src_hbm, dst_hbm, zd_hbm,
         acc_out, y_sh, acc_sh, src_v, dst_vA, dst_vB,
         r0, r1, r2, r3, g0, g1, g2, g3, s0, s1, s2, s3) = refs
    rows = (r0, r1, r2, r3)
    gs = (g0, g1, g2, g3)
    ss = (s0, s1, s2, s3)
    c = lax.axis_index("c")
    s = lax.axis_index("s")

    pltpu.sync_copy(y_hbm.at[c, pl.ds(s * RPT, RPT)],
                    y_sh.at[pl.ds(s * RPT, RPT)])
    pltpu.sync_copy(zd_hbm, acc_sh.at[pl.ds(s * RPT, RPT)])
    if with_deg:
        pltpu.sync_copy(z16_hbm, deg_sh.at[pl.ds(s * RPT, RPT)])
        pltpu.sync_copy(ones_hbm, ones_v)
    plsc.subcore_barrier()

    row_base = s * R_PER_T
    NBUF = 4
    LAG = 2

    def _wait_scatter(k):
        pltpu.make_async_copy(rows[k], acc_sh.at[dst_vA.at[0]], ss[k]).wait()

    def _wait_deg(par):
        pltpu.make_async_copy(ones_v, deg_sh.at[dst_vA.at[0]],
                              (d0, d1)[par]).wait()

    def _start_scatter(jj, dst_v, first):
        kk = jj % NBUF
        pltpu.make_async_copy(y_sh.at[src_v.at[jj]], rows[kk], gs[kk]).wait()
        pltpu.async_copy(rows[kk], acc_sh.at[dst_v.at[jj]], ss[kk], add=True)
        if with_deg:
            half = IDX_CH // 2
            mine = (c == 0) if jj < half else (c == 1)
            within = jj if jj < half else jj - half
            if within >= 2 or first is None:
                guard = mine
            else:
                guard = jnp.logical_and(mine, first)
            pl.when(guard)(lambda: _wait_deg(jj % 2))

            @pl.when(mine)
            def _(jj=jj):
                pltpu.async_copy(ones_v, deg_sh.at[dst_v.at[jj]],
                                 (d0, d1)[jj % 2], add=True)

    def _chunk(ci, dst_v, first):
        pltpu.sync_copy(src_hbm.at[pl.ds(row_base + ci * IDX_CH, IDX_CH)],
                        src_v)
        pltpu.sync_copy(dst_hbm.at[pl.ds(row_base + ci * IDX_CH, IDX_CH)],
                        dst_v)
        for j in range(IDX_CH):
            k = j % NBUF
            if j < NBUF:
                if first is None:
                    _wait_scatter(k)
                else:
                    pl.when(first)(lambda k=k: _wait_scatter(k))
            else:
                _wait_scatter(k)
            pltpu.async_copy(y_sh.at[src_v.at[j]], rows[k], gs[k])
            if j >= LAG:
                _start_scatter(j - LAG, dst_v, first)
        for jj in range(IDX_CH - LAG, IDX_CH):
            _start_scatter(jj, dst_v, first)

    def outer(cp, _):
        _chunk(2 * cp, dst_vA, cp > 0)
        _chunk(2 * cp + 1, dst_vB, None)
        return 0

    lax.fori_loop(0, R_PER_T // IDX_CH // 2, outer, 0)
    for k in range(NBUF):
        _wait_scatter(k)
    if with_deg:
        _wait_deg(0)
        _wait_deg(1)
    plsc.subcore_barrier()

    pltpu.sync_copy(acc_sh.at[pl.ds(s * RPT, RPT)],
                    acc_out.at[c, pl.ds(s * RPT, RPT)])
    if with_deg:
        pltpu.sync_copy(deg_sh.at[pl.ds(s * RPT, RPT)],
                        deg_out.at[c, pl.ds(s * RPT, RPT)])


_SC_PARAMS = pltpu.CompilerParams(use_tc_tiling_on_sc=False)

_sc_segsum = functools.partial(
    pl.kernel,
    functools.partial(_sc_segsum_body, False, IDX_CH),
    out_type=[jax.ShapeDtypeStruct((NC, N_PAD, DH), jnp.float32)],
    mesh=_MESH,
    scratch_types=(
        [pltpu.VMEM_SHARED((N_PAD, DH), jnp.float32)] * 2
        + [pltpu.VMEM((IDX_CH, LANES), jnp.int32)] * 3
        + [pltpu.VMEM((LANES, DH), jnp.float32)] * 4
        + [pltpu.SemaphoreType.DMA] * 8
    ),
    compiler_params=_SC_PARAMS,
)()

_sc_segsum_deg = functools.partial(
    pl.kernel,
    functools.partial(_sc_segsum_body, True, IDX_CH_DEG),
    out_type=[jax.ShapeDtypeStruct((NC, N_PAD, DH), jnp.float32),
              jax.ShapeDtypeStruct((NC, N_PAD, 16), jnp.float32)],
    mesh=_MESH,
    scratch_types=(
        [pltpu.VMEM_SHARED((N_PAD, DH), jnp.float32)] * 2
        + [pltpu.VMEM_SHARED((N_PAD, 16), jnp.float32)]
        + [pltpu.VMEM((IDX_CH_DEG, LANES), jnp.int32)] * 3
        + [pltpu.VMEM((LANES, DH), jnp.float32)] * 4
        + [pltpu.SemaphoreType.DMA] * 8
        + [pltpu.VMEM((LANES, 16), jnp.float32)]
        + [pltpu.SemaphoreType.DMA] * 2
    ),
    compiler_params=_SC_PARAMS,
)()


def _split_cols(y_ref, y):
    y_ref[0] = y[:, :DH]
    y_ref[1] = y[:, DH:]


def _mm_first_body(x_ref, wlT_ref, wrT_ref, bl_ref, y_ref, z_ref):
    xb = x_ref[...]
    _split_cols(y_ref,
                jnp.dot(xb, wlT_ref[...], preferred_element_type=jnp.float32))
    z_ref[...] = (jnp.dot(xb, wrT_ref[...], preferred_element_type=jnp.float32)
                  + bl_ref[...])


def _mm_first(x, wlT, wrT, bl):
    return pl.pallas_call(
        _mm_first_body,
        grid=(N_PAD // BLK,),
        in_specs=[
            pl.BlockSpec((BLK, D), lambda i: (i, 0)),
            pl.BlockSpec((D, D), lambda i: (0, 0)),
            pl.BlockSpec((D, D), lambda i: (0, 0)),
            pl.BlockSpec((1, D), lambda i: (0, 0)),
        ],
        out_specs=[pl.BlockSpec((NC, BLK, DH), lambda i: (0, i, 0)),
                   pl.BlockSpec((BLK, D), lambda i: (i, 0))],
        out_shape=[jax.ShapeDtypeStruct((NC, N_PAD, DH), jnp.float32),
                   jax.ShapeDtypeStruct((N, D), jnp.float32)],
    )(x, wlT, wrT, bl)


def _combine(acc_ref, deg_ref, z_ref, g_ref, b_ref):
    accsum = jnp.concatenate([acc_ref[0], acc_ref[1]], axis=1)
    deg = deg_ref[0, :, 0:1] + deg_ref[1, :, 0:1]
    h = accsum / jnp.maximum(deg, 1.0) + z_ref[...]
    h = h * (g_ref[...] * BN_RSQRT) + b_ref[...]
    return jnp.maximum(h, 0.0)


def _combine_mm_body(acc_ref, deg_ref, z_ref, g_ref, b_ref, wlT_ref, wrT_ref,
                     bln_ref, y_ref, zn_ref):
    h = _combine(acc_ref, deg_ref, z_ref, g_ref, b_ref)
    _split_cols(y_ref,
                jnp.dot(h, wlT_ref[...], preferred_element_type=jnp.float32))
    zn_ref[...] = (jnp.dot(h, wrT_ref[...], preferred_element_type=jnp.float32)
                   + bln_ref[...])


def _combine_mm(acc, deg, z, g, b, wlT, wrT, bln):
    return pl.pallas_call(
        _combine_mm_body,
        grid=(N_PAD // BLK,),
        in_specs=[
            pl.BlockSpec((NC, BLK, DH), lambda i: (0, i, 0)),
            pl.BlockSpec((NC, BLK, 16), lambda i: (0, i, 0)),
            pl.BlockSpec((BLK, D), lambda i: (i, 0)),
            pl.BlockSpec((1, D), lambda i: (0, 0)),
            pl.BlockSpec((1, D), lambda i: (0, 0)),
            pl.BlockSpec((D, D), lambda i: (0, 0)),
            pl.BlockSpec((D, D), lambda i: (0, 0)),
            pl.BlockSpec((1, D), lambda i: (0, 0)),
        ],
        out_specs=[pl.BlockSpec((NC, BLK, DH), lambda i: (0, i, 0)),
                   pl.BlockSpec((BLK, D), lambda i: (i, 0))],
        out_shape=[jax.ShapeDtypeStruct((NC, N_PAD, DH), jnp.float32),
                   jax.ShapeDtypeStruct((N, D), jnp.float32)],
    )(acc, deg, z, g, b, wlT, wrT, bln)


def _combine_final_body(acc_ref, deg_ref, z_ref, h_ref):
    accsum = jnp.concatenate([acc_ref[0], acc_ref[1]], axis=1)
    deg = deg_ref[0, :, 0:1] + deg_ref[1, :, 0:1]
    h_ref[...] = accsum / jnp.maximum(deg, 1.0) + z_ref[...]


def _combine_final(acc, deg, z):
    return pl.pallas_call(
        _combine_final_body,
        grid=(N_PAD // BLK,),
        in_specs=[
            pl.BlockSpec((NC, BLK, DH), lambda i: (0, i, 0)),
            pl.BlockSpec((NC, BLK, 16), lambda i: (0, i, 0)),
            pl.BlockSpec((BLK, D), lambda i: (i, 0)),
        ],
        out_specs=pl.BlockSpec((BLK, D), lambda i: (i, 0)),
        out_shape=jax.ShapeDtypeStruct((N, D), jnp.float32),
    )(acc, deg, z)


def kernel(x, edge_index, Wl1, bl1, Wr1, g1, b1, Wl2, bl2, Wr2, g2, b2,
           Wl3, bl3, Wr3):
    src = edge_index[0].astype(jnp.int32)
    dst = edge_index[1].astype(jnp.int32)
    pad = E_ROWS_PAD * LANES - E
    src2 = jnp.concatenate([src, jnp.zeros((pad,), jnp.int32)])
    src2 = src2.reshape(E_ROWS_PAD, LANES)
    dst2 = jnp.concatenate([dst, jnp.full((pad,), N, jnp.int32)])
    dst2 = dst2.reshape(E_ROWS_PAD, LANES)

    zdh = jnp.zeros((RPT, DH), jnp.float32)
    z16 = jnp.zeros((RPT, 16), jnp.float32)
    ones16 = jnp.ones((LANES, 16), jnp.float32)

    bl1r = bl1.reshape(1, D)
    bl2r = bl2.reshape(1, D)
    bl3r = bl3.reshape(1, D)
    g1r, b1r = g1.reshape(1, D), b1.reshape(1, D)
    g2r, b2r = g2.reshape(1, D), b2.reshape(1, D)

    y1, z1 = _mm_first(x, Wl1.T, Wr1.T, bl1r)
    acc1, deg = _sc_segsum_deg(y1, src2, dst2, zdh, z16, ones16)
    y2, z2 = _combine_mm(acc1, deg, z1, g1r, b1r, Wl2.T, Wr2.T, bl2r)
    acc2 = _sc_segsum(y2, src2, dst2, zdh)[0]
    y3, z3 = _combine_mm(acc2, deg, z2, g2r, b2r, Wl3.T, Wr3.T, bl3r)
    acc3 = _sc_segsum(y3, src2, dst2, zdh)[0]
    return _combine_final(acc3, deg, z3)


# speedup vs baseline: 1.0355x; 1.0355x over previous
import functools

import jax
import jax.numpy as jnp
from jax import lax
from jax.experimental import pallas as pl
from jax.experimental.pallas import tpu as pltpu
from jax.experimental.pallas import tpu_sc as plsc

N = 10000
E = 320000
D = 128
DH = D // 2
LANES = 128
NC = 2
NS = 16
NW = NC * NS
E_ROWS = -(-E // LANES)
E_ROWS_PAD = -(-E_ROWS // (NW * 8)) * NW * 8
R_PER_W = E_ROWS_PAD // NW
R_PER_T = E_ROWS_PAD // NS
N_PAD = 10112
RPT = N_PAD // NS
IDX_CH = 16
IDX_CH_DEG = 8
BLK = N_PAD // 16
BN_RSQRT = 1.0 / (1.0 + 1e-5) ** 0.5

_MESH = plsc.VectorSubcoreMesh(core_axis_name="c", subcore_axis_name="s")


def _sc_segsum_body(with_deg, IDX_CH, *refs):
    if with_deg:
        (y_hbm, src_hbm, dst_hbm, zd_hbm, z16_hbm, ones_hbm,
         acc_out, deg_out, y_sh, acc_sh, deg_sh, src_v, dst_vA, dst_vB,
         r0, r1, r2, r3, g0, g1, g2, g3, s0, s1, s2, s3, ones_v,
         d0, d1) = refs
    else:
        (y_hbm, src_hbm, dst_hbm, zd_hbm,
         acc_out, y_sh, acc_sh, src_v, dst_vA, dst_vB,
         r0, r1, r2, r3, g0, g1, g2, g3, s0, s1, s2, s3) = refs
    rows = (r0, r1, r2, r3)
    gs = (g0, g1, g2, g3)
    ss = (s0, s1, s2, s3)
    c = lax.axis_index("c")
    s = lax.axis_index("s")

    pltpu.sync_copy(y_hbm.at[c, pl.ds(s * RPT, RPT)],
                    y_sh.at[pl.ds(s * RPT, RPT)])
    pltpu.sync_copy(zd_hbm, acc_sh.at[pl.ds(s * RPT, RPT)])
    if with_deg:
        pltpu.sync_copy(z16_hbm, deg_sh.at[pl.ds(s * RPT, RPT)])
        pltpu.sync_copy(ones_hbm, ones_v)
    plsc.subcore_barrier()

    row_base = s * R_PER_T
    NBUF = 4
    LAG = 2

    def _wait_scatter(k):
        pltpu.make_async_copy(rows[k], acc_sh.at[dst_vA.at[0]], ss[k]).wait()

    def _wait_deg(par):
        pltpu.make_async_copy(ones_v, deg_sh.at[dst_vA.at[0]],
                              (d0, d1)[par]).wait()

    def _start_scatter(jj, dst_v, first):
        kk = jj % NBUF
        pltpu.make_async_copy(y_sh.at[src_v.at[jj]], rows[kk], gs[kk]).wait()
        pltpu.async_copy(rows[kk], acc_sh.at[dst_v.at[jj]], ss[kk], add=True)
        if with_deg:
            if jj >= 2:
                _wait_deg(jj % 2)
            elif first is None:
                _wait_deg(jj % 2)
            else:
                pl.when(first)(lambda: _wait_deg(jj % 2))
            pltpu.async_copy(ones_v, deg_sh.at[dst_v.at[jj]],
                             (d0, d1)[jj % 2], add=True)

    def _chunk(ci, dst_v, first):
        pltpu.sync_copy(src_hbm.at[pl.ds(row_base + ci * IDX_CH, IDX_CH)],
                        src_v)
        pltpu.sync_copy(dst_hbm.at[pl.ds(row_base + ci * IDX_CH, IDX_CH)],
                        dst_v)
        for j in range(IDX_CH):
            k = j % NBUF
            if j < NBUF:
                if first is None:
                    _wait_scatter(k)
                else:
                    pl.when(first)(lambda k=k: _wait_scatter(k))
            else:
                _wait_scatter(k)
            pltpu.async_copy(y_sh.at[src_v.at[j]], rows[k], gs[k])
            if j >= LAG:
                _start_scatter(j - LAG, dst_v, None if j - LAG >= 2 else first)
        for jj in range(IDX_CH - LAG, IDX_CH):
            _start_scatter(jj, dst_v, None)

    def outer(cp, _):
        _chunk(2 * cp, dst_vA, cp > 0)
        _chunk(2 * cp + 1, dst_vB, None)
        return 0

    lax.fori_loop(0, R_PER_T // IDX_CH // 2, outer, 0)
    for k in range(NBUF):
        _wait_scatter(k)
    if with_deg:
        _wait_deg(0)
        _wait_deg(1)
    plsc.subcore_barrier()

    pltpu.sync_copy(acc_sh.at[pl.ds(s * RPT, RPT)],
                    acc_out.at[c, pl.ds(s * RPT, RPT)])
    if with_deg:
        pltpu.sync_copy(deg_sh.at[pl.ds(s * RPT, RPT)],
                        deg_out.at[c, pl.ds(s * RPT, RPT)])


_SC_PARAMS = pltpu.CompilerParams(use_tc_tiling_on_sc=False)

_sc_segsum = functools.partial(
    pl.kernel,
    functools.partial(_sc_segsum_body, False, IDX_CH),
    out_type=[jax.ShapeDtypeStruct((NC, N_PAD, DH), jnp.float32)],
    mesh=_MESH,
    scratch_types=(
        [pltpu.VMEM_SHARED((N_PAD, DH), jnp.float32)] * 2
        + [pltpu.VMEM((IDX_CH, LANES), jnp.int32)] * 3
        + [pltpu.VMEM((LANES, DH), jnp.float32)] * 4
        + [pltpu.SemaphoreType.DMA] * 8
    ),
    compiler_params=_SC_PARAMS,
)()

_sc_segsum_deg = functools.partial(
    pl.kernel,
    functools.partial(_sc_segsum_body, True, IDX_CH_DEG),
    out_type=[jax.ShapeDtypeStruct((NC, N_PAD, DH), jnp.float32),
              jax.ShapeDtypeStruct((NC, N_PAD, 16), jnp.float32)],
    mesh=_MESH,
    scratch_types=(
        [pltpu.VMEM_SHARED((N_PAD, DH), jnp.float32)] * 2
        + [pltpu.VMEM_SHARED((N_PAD, 16), jnp.float32)]
        + [pltpu.VMEM((IDX_CH_DEG, LANES), jnp.int32)] * 3
        + [pltpu.VMEM((LANES, DH), jnp.float32)] * 4
        + [pltpu.SemaphoreType.DMA] * 8
        + [pltpu.VMEM((LANES, 16), jnp.float32)]
        + [pltpu.SemaphoreType.DMA] * 2
    ),
    compiler_params=_SC_PARAMS,
)()


def _split_cols(y_ref, y):
    y_ref[0] = y[:, :DH]
    y_ref[1] = y[:, DH:]


def _mm_first_body(x_ref, wlT_ref, wrT_ref, bl_ref, y_ref, z_ref):
    xb = x_ref[...]
    _split_cols(y_ref,
                jnp.dot(xb, wlT_ref[...], preferred_element_type=jnp.float32))
    z_ref[...] = (jnp.dot(xb, wrT_ref[...], preferred_element_type=jnp.float32)
                  + bl_ref[...])


def _mm_first(x, wlT, wrT, bl):
    return pl.pallas_call(
        _mm_first_body,
        grid=(N_PAD // BLK,),
        in_specs=[
            pl.BlockSpec((BLK, D), lambda i: (i, 0)),
            pl.BlockSpec((D, D), lambda i: (0, 0)),
            pl.BlockSpec((D, D), lambda i: (0, 0)),
            pl.BlockSpec((1, D), lambda i: (0, 0)),
        ],
        out_specs=[pl.BlockSpec((NC, BLK, DH), lambda i: (0, i, 0)),
                   pl.BlockSpec((BLK, D), lambda i: (i, 0))],
        out_shape=[jax.ShapeDtypeStruct((NC, N_PAD, DH), jnp.float32),
                   jax.ShapeDtypeStruct((N, D), jnp.float32)],
    )(x, wlT, wrT, bl)


def _combine(acc_ref, deg_ref, z_ref, g_ref, b_ref):
    accsum = jnp.concatenate([acc_ref[0], acc_ref[1]], axis=1)
    deg = deg_ref[0, :, 0:1]
    h = accsum / jnp.maximum(deg, 1.0) + z_ref[...]
    h = h * (g_ref[...] * BN_RSQRT) + b_ref[...]
    return jnp.maximum(h, 0.0)


def _combine_mm_body(acc_ref, deg_ref, z_ref, g_ref, b_ref, wlT_ref, wrT_ref,
                     bln_ref, y_ref, zn_ref):
    h = _combine(acc_ref, deg_ref, z_ref, g_ref, b_ref)
    _split_cols(y_ref,
                jnp.dot(h, wlT_ref[...], preferred_element_type=jnp.float32))
    zn_ref[...] = (jnp.dot(h, wrT_ref[...], preferred_element_type=jnp.float32)
                   + bln_ref[...])


def _combine_mm(acc, deg, z, g, b, wlT, wrT, bln):
    return pl.pallas_call(
        _combine_mm_body,
        grid=(N_PAD // BLK,),
        in_specs=[
            pl.BlockSpec((NC, BLK, DH), lambda i: (0, i, 0)),
            pl.BlockSpec((NC, BLK, 16), lambda i: (0, i, 0)),
            pl.BlockSpec((BLK, D), lambda i: (i, 0)),
            pl.BlockSpec((1, D), lambda i: (0, 0)),
            pl.BlockSpec((1, D), lambda i: (0, 0)),
            pl.BlockSpec((D, D), lambda i: (0, 0)),
            pl.BlockSpec((D, D), lambda i: (0, 0)),
            pl.BlockSpec((1, D), lambda i: (0, 0)),
        ],
        out_specs=[pl.BlockSpec((NC, BLK, DH), lambda i: (0, i, 0)),
                   pl.BlockSpec((BLK, D), lambda i: (i, 0))],
        out_shape=[jax.ShapeDtypeStruct((NC, N_PAD, DH), jnp.float32),
                   jax.ShapeDtypeStruct((N, D), jnp.float32)],
    )(acc, deg, z, g, b, wlT, wrT, bln)


def _combine_final_body(acc_ref, deg_ref, z_ref, h_ref):
    accsum = jnp.concatenate([acc_ref[0], acc_ref[1]], axis=1)
    deg = deg_ref[0, :, 0:1]
    h_ref[...] = accsum / jnp.maximum(deg, 1.0) + z_ref[...]


def _combine_final(acc, deg, z):
    return pl.pallas_call(
        _combine_final_body,
        grid=(N_PAD // BLK,),
        in_specs=[
            pl.BlockSpec((NC, BLK, DH), lambda i: (0, i, 0)),
            pl.BlockSpec((NC, BLK, 16), lambda i: (0, i, 0)),
            pl.BlockSpec((BLK, D), lambda i: (i, 0)),
        ],
        out_specs=pl.BlockSpec((BLK, D), lambda i: (i, 0)),
        out_shape=jax.ShapeDtypeStruct((N, D), jnp.float32),
    )(acc, deg, z)


def kernel(x, edge_index, Wl1, bl1, Wr1, g1, b1, Wl2, bl2, Wr2, g2, b2,
           Wl3, bl3, Wr3):
    src = edge_index[0].astype(jnp.int32)
    dst = edge_index[1].astype(jnp.int32)
    pad = E_ROWS_PAD * LANES - E
    src2 = jnp.concatenate([src, jnp.zeros((pad,), jnp.int32)])
    src2 = src2.reshape(E_ROWS_PAD, LANES)
    dst2 = jnp.concatenate([dst, jnp.full((pad,), N, jnp.int32)])
    dst2 = dst2.reshape(E_ROWS_PAD, LANES)

    zdh = jnp.zeros((RPT, DH), jnp.float32)
    z16 = jnp.zeros((RPT, 16), jnp.float32)
    ones16 = jnp.ones((LANES, 16), jnp.float32)

    bl1r = bl1.reshape(1, D)
    bl2r = bl2.reshape(1, D)
    bl3r = bl3.reshape(1, D)
    g1r, b1r = g1.reshape(1, D), b1.reshape(1, D)
    g2r, b2r = g2.reshape(1, D), b2.reshape(1, D)

    y1, z1 = _mm_first(x, Wl1.T, Wr1.T, bl1r)
    acc1, deg = _sc_segsum_deg(y1, src2, dst2, zdh, z16, ones16)
    y2, z2 = _combine_mm(acc1, deg, z1, g1r, b1r, Wl2.T, Wr2.T, bl2r)
    acc2 = _sc_segsum(y2, src2, dst2, zdh)[0]
    y3, z3 = _combine_mm(acc2, deg, z2, g2r, b2r, Wl3.T, Wr3.T, bl3r)
    acc3 = _sc_segsum(y3, src2, dst2, zdh)[0]
    return _combine_final(acc3, deg, z3)
